# baseline (speedup 1.0000x reference)
import functools

import jax
import jax.numpy as jnp
from jax.experimental import pallas as pl


def _block(n: int, cap: int) -> int:
    for b in range(min(cap, n), 7, -1):
        if n % b == 0 and b % 8 == 0:
            return b
    return n


def _gin_kernel(g_ref, xs_ref, xfull_ref, w1_ref, b1_ref, w2_ref, b2_ref,
                out_ref, *, n_i: int):
    i = pl.program_id(0)

    @pl.when(i == 0)
    def _init():
        out_ref[...] = jnp.zeros_like(out_ref)

    mask = (g_ref[...] > 0).astype(jnp.bfloat16)
    x = xs_ref[...].astype(jnp.bfloat16)
    out_ref[...] += jax.lax.dot_general(
        mask, x, (((0,), (0,)), ((), ())),
        preferred_element_type=jnp.float32)

    @pl.when(i == n_i - 1)
    def _epilogue():
        pre = xfull_ref[...] + out_ref[...]
        hid = jnp.maximum(
            jnp.dot(pre, w1_ref[...], preferred_element_type=jnp.float32)
            + b1_ref[...], 0.0)
        out_ref[...] = jnp.maximum(
            jnp.dot(hid, w2_ref[...], preferred_element_type=jnp.float32)
            + b2_ref[...], 0.0)


@jax.jit
def kernel(g, h, W1, b1, W2, b2):
    n, d = h.shape
    bi = _block(n, 200)
    n_i = n // bi
    b1r = b1.reshape(1, d)
    b2r = b2.reshape(1, d)
    return pl.pallas_call(
        functools.partial(_gin_kernel, n_i=n_i),
        grid=(n_i,),
        in_specs=[
            pl.BlockSpec((bi, n), lambda i: (i, 0)),
            pl.BlockSpec((bi, d), lambda i: (i, 0)),
            pl.BlockSpec((n, d), lambda i: (0, 0)),
            pl.BlockSpec((d, d), lambda i: (0, 0)),
            pl.BlockSpec((1, d), lambda i: (0, 0)),
            pl.BlockSpec((d, d), lambda i: (0, 0)),
            pl.BlockSpec((1, d), lambda i: (0, 0)),
        ],
        out_specs=pl.BlockSpec((n, d), lambda i: (0, 0)),
        out_shape=jax.ShapeDtypeStruct((n, d), jnp.float32),
    )(g, h, h, W1, b1r, W2, b2r)


# speedup vs baseline: 1.0603x; 1.0603x over previous
import functools

import jax
import jax.numpy as jnp
from jax.experimental import pallas as pl


def _block(n: int, cap: int) -> int:
    for b in range(min(cap, n), 7, -1):
        if n % b == 0 and b % 8 == 0:
            return b
    return n


def _gin_kernel(g_ref, xs_ref, xfull_ref, w1_ref, b1_ref, w2_ref, b2_ref,
                out_ref, *, n_i: int):
    i = pl.program_id(0)

    @pl.when(i == 0)
    def _init():
        out_ref[...] = jnp.zeros_like(out_ref)

    mask = (g_ref[...] > 0).astype(jnp.bfloat16)
    x = xs_ref[...].astype(jnp.bfloat16)
    out_ref[...] += jax.lax.dot_general(
        mask, x, (((0,), (0,)), ((), ())),
        preferred_element_type=jnp.float32)

    @pl.when(i == n_i - 1)
    def _epilogue():
        pre = xfull_ref[...] + out_ref[...]
        hid = jnp.maximum(
            jnp.dot(pre, w1_ref[...], preferred_element_type=jnp.float32)
            + b1_ref[...], 0.0)
        out_ref[...] = jnp.maximum(
            jnp.dot(hid, w2_ref[...], preferred_element_type=jnp.float32)
            + b2_ref[...], 0.0)


@jax.jit
def kernel(g, h, W1, b1, W2, b2):
    n, d = h.shape
    bi = _block(n, 400)
    n_i = n // bi
    b1r = b1.reshape(1, d)
    b2r = b2.reshape(1, d)
    return pl.pallas_call(
        functools.partial(_gin_kernel, n_i=n_i),
        grid=(n_i,),
        in_specs=[
            pl.BlockSpec((bi, n), lambda i: (i, 0)),
            pl.BlockSpec((bi, d), lambda i: (i, 0)),
            pl.BlockSpec((n, d), lambda i: (0, 0)),
            pl.BlockSpec((d, d), lambda i: (0, 0)),
            pl.BlockSpec((1, d), lambda i: (0, 0)),
            pl.BlockSpec((d, d), lambda i: (0, 0)),
            pl.BlockSpec((1, d), lambda i: (0, 0)),
        ],
        out_specs=pl.BlockSpec((n, d), lambda i: (0, 0)),
        out_shape=jax.ShapeDtypeStruct((n, d), jnp.float32),
    )(g, h, h, W1, b1r, W2, b2r)
